# baseline (speedup 1.0000x reference)
import jax
import jax.numpy as jnp
from jax.experimental import pallas as pl
from jax.experimental.pallas import tpu as pltpu

_BI = 512


def _fused(adj_ref, seq_ref, w_ref, bias_ref, a_ref, out_ref,
           pack_ref, sf_ref, h_ref):
    g = pl.program_id(0)
    n = h_ref.shape[0]
    nh = n // 2
    ni = n // _BI

    @pl.when(g == 0)
    def _init():
        sf_ref[...] = jax.lax.dot_general(
            seq_ref[...], w_ref[...],
            (((1,), (1,)), ((), ())),
            preferred_element_type=jnp.float32,
        )
        h_ref[...] = jnp.zeros_like(h_ref)

    @pl.when(g < ni)
    def _stream_step():
        rows = pl.ds(g * _BI, _BI)
        blk = adj_ref[...]

        hk = jax.lax.dot_general(
            blk, sf_ref[...],
            (((1,), (0,)), ((), ())),
            preferred_element_type=jnp.float32,
        ) + bias_ref[...]

        for j in range(ni):
            @pl.when(j >= g)
            def _pack_tile(j=j):
                base = (j * (j + 1) // 2) * _BI
                pack_ref[pl.ds(base + g * _BI, _BI), :] = (
                    blk[:, j * _BI:(j + 1) * _BI].astype(jnp.bfloat16))

        out_ref[rows, :] = jax.lax.dot_general(
            blk[:, :nh], h_ref[:nh, :],
            (((1,), (0,)), ((), ())),
            preferred_element_type=jnp.float32,
        )

        @pl.when(g * _BI > nh)
        def _dota_hi():
            out_ref[rows, :] += jax.lax.dot_general(
                blk[:, nh:], h_ref[nh:, :],
                (((1,), (0,)), ((), ())),
                preferred_element_type=jnp.float32,
            )

        hkb = hk.astype(jnp.bfloat16)
        h_ref[rows, :] = hk

        pbase = (g * (g + 1) // 2) * _BI
        for q in range(ni):
            @pl.when(q <= g)
            def _dotb_chunk(q=q):
                out_ref[pl.ds(q * _BI, _BI), :] += jax.lax.dot_general(
                    pack_ref[pl.ds(pbase + q * _BI, _BI), :], hkb,
                    (((1,), (0,)), ((), ())),
                    preferred_element_type=jnp.float32,
                )

    @pl.when(g == ni)
    def _epilogue():
        o = out_ref[...] + bias_ref[...]
        out_ref[...] = jnp.where(o > 0, o, a_ref[0, 0] * o)


def kernel(seq, adj, du, W, bias, prelu_a):
    del du
    (b, n, f_in) = seq.shape
    f_out = W.shape[0]
    seq2 = seq.reshape(n, f_in).astype(jnp.bfloat16)
    adj2 = adj.reshape(n, n)
    bias2 = bias.reshape(1, f_out)
    a2 = jnp.reshape(prelu_a, (1, 1)).astype(jnp.float32)

    ni = n // _BI
    ntri = ni * (ni + 1) // 2

    out = pl.pallas_call(
        _fused,
        grid=(ni + 1,),
        in_specs=[
            pl.BlockSpec((_BI, n), lambda g: (jnp.minimum(g, ni - 1), 0)),
            pl.BlockSpec((n, f_in), lambda g: (0, 0)),
            pl.BlockSpec((f_out, f_in), lambda g: (0, 0)),
            pl.BlockSpec((1, f_out), lambda g: (0, 0)),
            pl.BlockSpec((1, 1), lambda g: (0, 0)),
        ],
        out_specs=pl.BlockSpec((n, f_out), lambda g: (0, 0)),
        out_shape=jax.ShapeDtypeStruct((n, f_out), jnp.float32),
        scratch_shapes=[
            pltpu.VMEM((ntri * _BI, _BI), jnp.bfloat16),
            pltpu.VMEM((n, f_out), jnp.float32),
            pltpu.VMEM((n, f_out), jnp.float32),
        ],
        compiler_params=pltpu.CompilerParams(
            vmem_limit_bytes=64 * 1024 * 1024,
        ),
    )(adj2, seq2, W.astype(jnp.bfloat16), bias2, a2)

    return out.reshape(b, n, f_out)


# speedup vs baseline: 1.3515x; 1.3515x over previous
import jax
import jax.numpy as jnp
from jax.experimental import pallas as pl
from jax.experimental.pallas import tpu as pltpu

_BI = 512
_BO = 1024


def _fused(adj_ref, seq_ref, w_ref, bias_ref, a_ref, out_ref,
           adjbf_ref, sf_ref, h_ref):
    g = pl.program_id(0)
    n = adjbf_ref.shape[0]
    ni = n // _BI

    @pl.when(g == 0)
    def _compute_sf():
        sf_ref[...] = jax.lax.dot_general(
            seq_ref[...], w_ref[...],
            (((1,), (1,)), ((), ())),
            preferred_element_type=jnp.float32,
        )

    @pl.when(g < ni)
    def _phase0():
        rows = pl.ds(g * _BI, _BI)
        blk = adj_ref[...]
        adjbf_ref[rows, :] = blk.astype(jnp.bfloat16)
        h = jax.lax.dot_general(
            blk, sf_ref[...],
            (((1,), (0,)), ((), ())),
            preferred_element_type=jnp.float32,
        ) + bias_ref[...]
        h_ref[rows, :] = h.astype(jnp.bfloat16)

    @pl.when(g >= ni)
    def _phase1():
        rows = pl.ds((g - ni) * _BO, _BO)
        o = jax.lax.dot_general(
            adjbf_ref[rows, :], h_ref[...],
            (((1,), (0,)), ((), ())),
            preferred_element_type=jnp.float32,
        ) + bias_ref[...]
        out_ref[...] = jnp.where(o > 0, o, a_ref[0, 0] * o)


def kernel(seq, adj, du, W, bias, prelu_a):
    del du
    (b, n, f_in) = seq.shape
    f_out = W.shape[0]
    seq2 = seq.reshape(n, f_in).astype(jnp.bfloat16)
    adj2 = adj.reshape(n, n)
    bias2 = bias.reshape(1, f_out)
    a2 = jnp.reshape(prelu_a, (1, 1)).astype(jnp.float32)

    ni = n // _BI
    no = n // _BO

    out = pl.pallas_call(
        _fused,
        grid=(ni + no,),
        in_specs=[
            pl.BlockSpec((_BI, n), lambda g: (jnp.minimum(g, ni - 1), 0)),
            pl.BlockSpec((n, f_in), lambda g: (0, 0)),
            pl.BlockSpec((f_out, f_in), lambda g: (0, 0)),
            pl.BlockSpec((1, f_out), lambda g: (0, 0)),
            pl.BlockSpec((1, 1), lambda g: (0, 0)),
        ],
        out_specs=pl.BlockSpec(
            (_BO, f_out), lambda g: (jnp.maximum(g - ni, 0), 0)),
        out_shape=jax.ShapeDtypeStruct((n, f_out), jnp.float32),
        scratch_shapes=[
            pltpu.VMEM((n, n), jnp.bfloat16),
            pltpu.VMEM((n, f_out), jnp.float32),
            pltpu.VMEM((n, f_out), jnp.bfloat16),
        ],
        compiler_params=pltpu.CompilerParams(
            vmem_limit_bytes=64 * 1024 * 1024,
        ),
    )(adj2, seq2, W.astype(jnp.bfloat16), bias2, a2)

    return out.reshape(b, n, f_out)


# speedup vs baseline: 1.5273x; 1.1301x over previous
import jax
import jax.numpy as jnp
from jax.experimental import pallas as pl
from jax.experimental.pallas import tpu as pltpu

_BI = 512
_BO = 1024


def _fused(adj_ref, seq_ref, w_ref, bias_ref, a_ref, out_ref,
           adjbf_ref, sf_ref, h_ref):
    g = pl.program_id(0)
    n = adjbf_ref.shape[0]
    ni = n // _BI

    @pl.when(g == 0)
    def _compute_sf():
        sf_ref[...] = jax.lax.dot_general(
            seq_ref[...], w_ref[...],
            (((1,), (1,)), ((), ())),
            preferred_element_type=jnp.float32,
        )

    @pl.when(g < ni)
    def _phase0():
        rows = pl.ds(g * _BI, _BI)
        blk = adj_ref[...]
        adjbf_ref[rows, :] = blk.astype(jnp.bfloat16)
        h = jax.lax.dot_general(
            blk, sf_ref[...],
            (((1,), (0,)), ((), ())),
            preferred_element_type=jnp.float32,
        ) + bias_ref[...]
        h_ref[rows, :] = h.astype(jnp.bfloat16)

    @pl.when(g >= ni)
    def _phase1():
        rows = pl.ds((g - ni) * _BO, _BO)
        o = jax.lax.dot_general(
            adjbf_ref[rows, :], h_ref[...],
            (((1,), (0,)), ((), ())),
            preferred_element_type=jnp.float32,
        ) + bias_ref[...]
        out_ref[...] = jnp.where(o > 0, o, a_ref[0, 0] * o)


def kernel(seq, adj, du, W, bias, prelu_a):
    del du
    (b, n, f_in) = seq.shape
    f_out = W.shape[0]
    seq2 = seq.reshape(n, f_in)
    adj2 = adj.reshape(n, n)
    bias2 = bias.reshape(1, f_out)
    a2 = jnp.reshape(prelu_a, (1, 1)).astype(jnp.float32)

    ni = n // _BI
    no = n // _BO

    out = pl.pallas_call(
        _fused,
        grid=(ni + no,),
        in_specs=[
            pl.BlockSpec((_BI, n), lambda g: (jnp.minimum(g, ni - 1), 0)),
            pl.BlockSpec((n, f_in), lambda g: (0, 0)),
            pl.BlockSpec((f_out, f_in), lambda g: (0, 0)),
            pl.BlockSpec((1, f_out), lambda g: (0, 0)),
            pl.BlockSpec((1, 1), lambda g: (0, 0)),
        ],
        out_specs=pl.BlockSpec(
            (_BO, f_out), lambda g: (jnp.maximum(g - ni, 0), 0)),
        out_shape=jax.ShapeDtypeStruct((n, f_out), jnp.float32),
        scratch_shapes=[
            pltpu.VMEM((n, n), jnp.bfloat16),
            pltpu.VMEM((n, f_out), jnp.float32),
            pltpu.VMEM((n, f_out), jnp.bfloat16),
        ],
        compiler_params=pltpu.CompilerParams(
            vmem_limit_bytes=64 * 1024 * 1024,
        ),
    )(adj2, seq2, W, bias2, a2)

    return out.reshape(b, n, f_out)
